# Initial kernel scaffold; baseline (speedup 1.0000x reference)
import jax
import jax.numpy as jnp
from jax.experimental import pallas as pl


def kernel(x, edge_index, W1, b1, W2, b2, W3, b3):
    raise NotImplementedError("write your pallas kernel here")


# speedup vs baseline: 4.7040x; 4.7040x over previous
import functools

import jax
import jax.numpy as jnp
from jax import lax
from jax.experimental import pallas as pl
from jax.experimental.pallas import tpu as pltpu
from jax.experimental.pallas import tpu_sc as plsc

NC = 2
NS = 16
NW = NC * NS
C = 128
RB = 512


def _sc_mesh():
    return plsc.VectorSubcoreMesh(
        core_axis_name="c", subcore_axis_name="s",
        num_cores=NC, num_subcores=NS)


def _unpack_chunk(pk_v, j, row_c, col_c):
    for t in range(C // 16):
        v = pk_v[j, pl.ds(t * 16, 16)]
        if row_c is not None:
            row_c[pl.ds(t * 16, 16)] = jnp.bitwise_and(v, 0xFFFF)
        col_c[pl.ds(t * 16, 16)] = lax.shift_right_logical(v, 16)


def _make_deg(NP, K, rpt):
    nz = rpt // C

    @functools.partial(
        pl.kernel,
        out_type=jax.ShapeDtypeStruct((NC, NP, 128), jnp.float32),
        mesh=_sc_mesh(),
        scratch_types=[
            pltpu.VMEM((C,), jnp.int32),
            pltpu.VMEM((C,), jnp.int32),
            pltpu.VMEM((C, 128), jnp.float32),
            pltpu.VMEM((C, 128), jnp.float32),
            pltpu.VMEM_SHARED((NP, 128), jnp.float32),
        ],
    )
    def deg_k(pk_hbm, deg_hbm, pk_c, col_c, ones_v, wb_v, acc):
        cid = lax.axis_index("c")
        sid = lax.axis_index("s")
        wid = sid * NC + cid

        def fill(i, _):
            for jj in range(8):
                ones_v[i, pl.ds(jj * 16, 16)] = jnp.ones((16,), jnp.float32)
                wb_v[i, pl.ds(jj * 16, 16)] = jnp.zeros((16,), jnp.float32)
            return 0
        lax.fori_loop(0, C, fill, 0)

        def zcopy(i, _):
            pltpu.sync_copy(wb_v, acc.at[pl.ds(sid * rpt + i * C, C)])
            return 0
        lax.fori_loop(0, nz, zcopy, 0)
        plsc.subcore_barrier()

        def body(j, _):
            pltpu.sync_copy(pk_hbm.at[wid, j], pk_c)
            for t in range(C // 16):
                v = pk_c[pl.ds(t * 16, 16)]
                col_c[pl.ds(t * 16, 16)] = lax.shift_right_logical(v, 16)
            pltpu.sync_copy(ones_v, acc.at[col_c], add=True)
            return 0
        lax.fori_loop(0, K, body, 0)
        plsc.subcore_barrier()

        def ocopy(i, _):
            base = sid * rpt + i * C
            pltpu.sync_copy(acc.at[pl.ds(base, C)], wb_v)
            pltpu.sync_copy(wb_v, deg_hbm.at[cid, pl.ds(base, C)])
            return 0
        lax.fori_loop(0, nz, ocopy, 0)

    return deg_k


def _make_agg(NP, D, K, rpt):
    nz = rpt // C

    @functools.partial(
        pl.kernel,
        out_type=jax.ShapeDtypeStruct((NC, NP, D), jnp.float32),
        mesh=_sc_mesh(),
        scratch_types=[
            pltpu.VMEM((K, C), jnp.int32),
            pltpu.VMEM((C,), jnp.int32),
            pltpu.VMEM((C,), jnp.int32),
            pltpu.VMEM((C, D), jnp.float32),
            pltpu.VMEM((C, D), jnp.float32),
            pltpu.VMEM_SHARED((NP, D), jnp.float32),
            pltpu.SemaphoreType.DMA,
            pltpu.SemaphoreType.DMA,
        ],
    )
    def agg_k(hhat_hbm, pk_hbm, out_hbm,
              pk_v, row_c, col_c, gb0, gb1, acc, sem0, sem1):
        cid = lax.axis_index("c")
        sid = lax.axis_index("s")
        wid = sid * NC + cid

        def zrow(i, _):
            for jj in range(D // 16):
                gb0[i, pl.ds(jj * 16, 16)] = jnp.zeros((16,), jnp.float32)
            return 0
        lax.fori_loop(0, C, zrow, 0)

        def zcopy(i, _):
            pltpu.sync_copy(gb0, acc.at[pl.ds(sid * rpt + i * C, C)])
            return 0
        lax.fori_loop(0, nz, zcopy, 0)

        pltpu.sync_copy(pk_hbm.at[wid], pk_v)
        plsc.subcore_barrier()

        def body(j, _):
            _unpack_chunk(pk_v, j, row_c, col_c)
            pltpu.async_copy(hhat_hbm.at[row_c], gb0, sem0).wait()
            pltpu.sync_copy(gb0, acc.at[col_c], add=True)
            return 0
        lax.fori_loop(0, K, body, 0)
        plsc.subcore_barrier()

        def ocopy(i, _):
            base = sid * rpt + i * C
            pltpu.sync_copy(acc.at[pl.ds(base, C)], gb1)
            pltpu.sync_copy(gb1, out_hbm.at[cid, pl.ds(base, C)])
            return 0
        lax.fori_loop(0, nz, ocopy, 0)

    return agg_k


def _dinv_block(dr):
    d = dr[0, :, 0:1] + dr[1, :, 0:1]
    return jnp.where(d > 0, lax.rsqrt(jnp.where(d > 0, d, 1.0)), 0.0)


def _tc_pre(x_pad, W, deg, NP, D):
    def body(xr, wr, dr, out):
        dinv = _dinv_block(dr)
        out[...] = dinv * jnp.dot(xr[...], wr[...],
                                  preferred_element_type=jnp.float32)

    return pl.pallas_call(
        body,
        grid=(NP // RB,),
        in_specs=[
            pl.BlockSpec((RB, D), lambda i: (i, 0)),
            pl.BlockSpec((D, D), lambda i: (0, 0)),
            pl.BlockSpec((NC, RB, 128), lambda i: (0, i, 0)),
        ],
        out_specs=pl.BlockSpec((RB, D), lambda i: (i, 0)),
        out_shape=jax.ShapeDtypeStruct((NP, D), jnp.float32),
    )(x_pad, W, deg)


def _tc_mid(s2, deg, b, Wn, NP, D):
    def body(sr, dr, br, wr, a_out, hn_out, hh_out):
        dinv = _dinv_block(dr)
        a = dinv * (sr[0] + sr[1]) + br[...]
        a_out[...] = a
        nsq = jnp.sum(a * a, axis=-1, keepdims=True)
        hn = a * lax.rsqrt(jnp.maximum(nsq, 1e-24))
        hn_out[...] = hn
        hh_out[...] = dinv * jnp.dot(hn, wr[...],
                                     preferred_element_type=jnp.float32)

    sds = jax.ShapeDtypeStruct((NP, D), jnp.float32)
    return pl.pallas_call(
        body,
        grid=(NP // RB,),
        in_specs=[
            pl.BlockSpec((NC, RB, D), lambda i: (0, i, 0)),
            pl.BlockSpec((NC, RB, 128), lambda i: (0, i, 0)),
            pl.BlockSpec((1, D), lambda i: (0, 0)),
            pl.BlockSpec((D, D), lambda i: (0, 0)),
        ],
        out_specs=[pl.BlockSpec((RB, D), lambda i: (i, 0))] * 3,
        out_shape=[sds, sds, sds],
    )(s2, deg, b, Wn)


def _tc_post(s2, deg, b, x_pad, hn1, hn2, a1, a2, NP, D):
    def body(sr, dr, br, xr, h1r, h2r, a1r, a2r, a3_out, res_out, fin_out):
        dinv = _dinv_block(dr)
        a3 = dinv * (sr[0] + sr[1]) + br[...]
        a3_out[...] = a3
        nsq = jnp.sum(a3 * a3, axis=-1, keepdims=True)
        hn3 = a3 * lax.rsqrt(jnp.maximum(nsq, 1e-24))
        res_out[...] = (xr[...] + h1r[...] + 0.5 * h2r[...]
                        + (1.0 / 3.0) * hn3)
        fin_out[...] = (a1r[...] + a2r[...] + a3) * (1.0 / 3.0)

    sds = jax.ShapeDtypeStruct((NP, D), jnp.float32)
    rspec = pl.BlockSpec((RB, D), lambda i: (i, 0))
    return pl.pallas_call(
        body,
        grid=(NP // RB,),
        in_specs=[
            pl.BlockSpec((NC, RB, D), lambda i: (0, i, 0)),
            pl.BlockSpec((NC, RB, 128), lambda i: (0, i, 0)),
            pl.BlockSpec((1, D), lambda i: (0, 0)),
            rspec, rspec, rspec, rspec, rspec,
        ],
        out_specs=[rspec] * 3,
        out_shape=[sds, sds, sds],
    )(s2, deg, b, x_pad, hn1, hn2, a1, a2)


def kernel(x, edge_index, W1, b1, W2, b2, W3, b3):
    info = plsc.get_sparse_core_info()
    if (info.num_cores, info.num_subcores, info.num_lanes) != (NC, NS, 16):
        raise ValueError(f"unexpected SC geometry: {info}")
    N, D = x.shape
    E = edge_index.shape[1]
    gran = NS * C
    NP = -(-(N + 1) // RB) * RB
    if NP % gran:
        NP = -(-NP // gran) * gran
    rpt = NP // NS
    K = 2 * (-(-E // (NW * C * 2)))
    Ep = NW * K * C

    row = edge_index[0]
    col = edge_index[1]
    pad = Ep - E
    if pad:
        padv = jnp.full((pad,), N, jnp.int32)
        row = jnp.concatenate([row, padv])
        col = jnp.concatenate([col, padv])
    packed = jnp.bitwise_or(row, jnp.left_shift(col, 16)).reshape(NW, K, C)
    x_pad = jnp.pad(x, ((0, NP - N), (0, 0)))
    b1r, b2r, b3r = (b.reshape(1, D) for b in (b1, b2, b3))

    deg = _make_deg(NP, K, rpt)(packed)
    agg = _make_agg(NP, D, K, rpt)

    hh1 = _tc_pre(x_pad, W1, deg, NP, D)
    s1 = agg(hh1, packed)
    a1, hn1, hh2 = _tc_mid(s1, deg, b1r, W2, NP, D)
    s2 = agg(hh2, packed)
    a2, hn2, hh3 = _tc_mid(s2, deg, b2r, W3, NP, D)
    s3 = agg(hh3, packed)
    a3, res_p, fin_p = _tc_post(s3, deg, b3r, x_pad, hn1, hn2, a1, a2, NP, D)
    return (res_p[:N], fin_p[:N], a3[:N])


# speedup vs baseline: 5.1730x; 1.0997x over previous
import functools

import jax
import jax.numpy as jnp
from jax import lax
from jax.experimental import pallas as pl
from jax.experimental.pallas import tpu as pltpu
from jax.experimental.pallas import tpu_sc as plsc

NC = 2
NS = 16
NW = NC * NS
C = 128
RB = 512


def _sc_mesh():
    return plsc.VectorSubcoreMesh(
        core_axis_name="c", subcore_axis_name="s",
        num_cores=NC, num_subcores=NS)


def _unpack_chunk(pk_v, j, row_c, col_c):
    for t in range(C // 16):
        v = pk_v[j, pl.ds(t * 16, 16)]
        if row_c is not None:
            row_c[pl.ds(t * 16, 16)] = jnp.bitwise_and(v, 0xFFFF)
        col_c[pl.ds(t * 16, 16)] = lax.shift_right_logical(v, 16)


def _make_deg(NP, K, rpt):
    nz = rpt // C

    @functools.partial(
        pl.kernel,
        out_type=jax.ShapeDtypeStruct((NC, NP, 128), jnp.float32),
        mesh=_sc_mesh(),
        scratch_types=[
            pltpu.VMEM((C,), jnp.int32),
            pltpu.VMEM((C,), jnp.int32),
            pltpu.VMEM((C, 128), jnp.float32),
            pltpu.VMEM((C, 128), jnp.float32),
            pltpu.VMEM_SHARED((NP, 128), jnp.float32),
        ],
    )
    def deg_k(pk_hbm, deg_hbm, pk_c, col_c, ones_v, wb_v, acc):
        cid = lax.axis_index("c")
        sid = lax.axis_index("s")
        wid = sid * NC + cid

        def fill(i, _):
            for jj in range(8):
                ones_v[i, pl.ds(jj * 16, 16)] = jnp.ones((16,), jnp.float32)
                wb_v[i, pl.ds(jj * 16, 16)] = jnp.zeros((16,), jnp.float32)
            return 0
        lax.fori_loop(0, C, fill, 0)

        def zcopy(i, _):
            pltpu.sync_copy(wb_v, acc.at[pl.ds(sid * rpt + i * C, C)])
            return 0
        lax.fori_loop(0, nz, zcopy, 0)
        plsc.subcore_barrier()

        def body(j, _):
            pltpu.sync_copy(pk_hbm.at[wid, j], pk_c)
            for t in range(C // 16):
                v = pk_c[pl.ds(t * 16, 16)]
                col_c[pl.ds(t * 16, 16)] = lax.shift_right_logical(v, 16)
            pltpu.sync_copy(ones_v, acc.at[col_c], add=True)
            return 0
        lax.fori_loop(0, K, body, 0)
        plsc.subcore_barrier()

        def ocopy(i, _):
            base = sid * rpt + i * C
            pltpu.sync_copy(acc.at[pl.ds(base, C)], wb_v)
            pltpu.sync_copy(wb_v, deg_hbm.at[cid, pl.ds(base, C)])
            return 0
        lax.fori_loop(0, nz, ocopy, 0)

    return deg_k


def _make_agg(NP, D, K, rpt):
    nz = rpt // C

    @functools.partial(
        pl.kernel,
        out_type=jax.ShapeDtypeStruct((NC, NP, D), jnp.float32),
        mesh=_sc_mesh(),
        scratch_types=[
            pltpu.VMEM((K, C), jnp.int32),
            pltpu.VMEM((C,), jnp.int32),
            pltpu.VMEM((C,), jnp.int32),
            pltpu.VMEM((C,), jnp.int32),
            pltpu.VMEM((C,), jnp.int32),
            pltpu.VMEM((C, D), jnp.float32),
            pltpu.VMEM((C, D), jnp.float32),
            pltpu.VMEM_SHARED((NP, D), jnp.float32),
            pltpu.SemaphoreType.DMA,
            pltpu.SemaphoreType.DMA,
        ],
    )
    def agg_k(hhat_hbm, pk_hbm, out_hbm,
              pk_v, row_c0, col_c0, row_c1, col_c1, gb0, gb1,
              acc, sem0, sem1):
        cid = lax.axis_index("c")
        sid = lax.axis_index("s")
        wid = sid * NC + cid
        rows = (row_c0, row_c1)
        cols = (col_c0, col_c1)
        gbufs = (gb0, gb1)
        sems = (sem0, sem1)

        def zrow(i, _):
            for jj in range(D // 16):
                gb0[i, pl.ds(jj * 16, 16)] = jnp.zeros((16,), jnp.float32)
            return 0
        lax.fori_loop(0, C, zrow, 0)

        def zcopy(i, _):
            pltpu.sync_copy(gb0, acc.at[pl.ds(sid * rpt + i * C, C)])
            return 0
        lax.fori_loop(0, nz, zcopy, 0)

        pltpu.sync_copy(pk_hbm.at[wid], pk_v)
        plsc.subcore_barrier()

        for b in (0, 1):
            _unpack_chunk(pk_v, b, rows[b], cols[b])
            pltpu.async_copy(hhat_hbm.at[rows[b]], gbufs[b], sems[b])

        def body(i, _):
            for b in (0, 1):
                j = 2 * i + b
                pltpu.make_async_copy(hhat_hbm.at[rows[b]], gbufs[b],
                                      sems[b]).wait()
                pltpu.sync_copy(gbufs[b], acc.at[cols[b]], add=True)
                _unpack_chunk(pk_v, j + 2, rows[b], cols[b])
                pltpu.async_copy(hhat_hbm.at[rows[b]], gbufs[b], sems[b])
            return 0
        lax.fori_loop(0, K // 2 - 1, body, 0)

        for b in (0, 1):
            pltpu.make_async_copy(hhat_hbm.at[rows[b]], gbufs[b],
                                  sems[b]).wait()
            pltpu.sync_copy(gbufs[b], acc.at[cols[b]], add=True)
        plsc.subcore_barrier()

        def ocopy(i, _):
            base = sid * rpt + i * C
            pltpu.sync_copy(acc.at[pl.ds(base, C)], gb1)
            pltpu.sync_copy(gb1, out_hbm.at[cid, pl.ds(base, C)])
            return 0
        lax.fori_loop(0, nz, ocopy, 0)

    return agg_k


def _dinv_block(dr):
    d = dr[0, :, 0:1] + dr[1, :, 0:1]
    return jnp.where(d > 0, lax.rsqrt(jnp.where(d > 0, d, 1.0)), 0.0)


def _tc_pre(x_pad, W, deg, NP, D):
    def body(xr, wr, dr, out):
        dinv = _dinv_block(dr)
        out[...] = dinv * jnp.dot(xr[...], wr[...],
                                  preferred_element_type=jnp.float32)

    return pl.pallas_call(
        body,
        grid=(NP // RB,),
        in_specs=[
            pl.BlockSpec((RB, D), lambda i: (i, 0)),
            pl.BlockSpec((D, D), lambda i: (0, 0)),
            pl.BlockSpec((NC, RB, 128), lambda i: (0, i, 0)),
        ],
        out_specs=pl.BlockSpec((RB, D), lambda i: (i, 0)),
        out_shape=jax.ShapeDtypeStruct((NP, D), jnp.float32),
    )(x_pad, W, deg)


def _tc_mid(s2, deg, b, Wn, NP, D):
    def body(sr, dr, br, wr, a_out, hn_out, hh_out):
        dinv = _dinv_block(dr)
        a = dinv * (sr[0] + sr[1]) + br[...]
        a_out[...] = a
        nsq = jnp.sum(a * a, axis=-1, keepdims=True)
        hn = a * lax.rsqrt(jnp.maximum(nsq, 1e-24))
        hn_out[...] = hn
        hh_out[...] = dinv * jnp.dot(hn, wr[...],
                                     preferred_element_type=jnp.float32)

    sds = jax.ShapeDtypeStruct((NP, D), jnp.float32)
    return pl.pallas_call(
        body,
        grid=(NP // RB,),
        in_specs=[
            pl.BlockSpec((NC, RB, D), lambda i: (0, i, 0)),
            pl.BlockSpec((NC, RB, 128), lambda i: (0, i, 0)),
            pl.BlockSpec((1, D), lambda i: (0, 0)),
            pl.BlockSpec((D, D), lambda i: (0, 0)),
        ],
        out_specs=[pl.BlockSpec((RB, D), lambda i: (i, 0))] * 3,
        out_shape=[sds, sds, sds],
    )(s2, deg, b, Wn)


def _tc_post(s2, deg, b, x_pad, hn1, hn2, a1, a2, NP, D):
    def body(sr, dr, br, xr, h1r, h2r, a1r, a2r, a3_out, res_out, fin_out):
        dinv = _dinv_block(dr)
        a3 = dinv * (sr[0] + sr[1]) + br[...]
        a3_out[...] = a3
        nsq = jnp.sum(a3 * a3, axis=-1, keepdims=True)
        hn3 = a3 * lax.rsqrt(jnp.maximum(nsq, 1e-24))
        res_out[...] = (xr[...] + h1r[...] + 0.5 * h2r[...]
                        + (1.0 / 3.0) * hn3)
        fin_out[...] = (a1r[...] + a2r[...] + a3) * (1.0 / 3.0)

    sds = jax.ShapeDtypeStruct((NP, D), jnp.float32)
    rspec = pl.BlockSpec((RB, D), lambda i: (i, 0))
    return pl.pallas_call(
        body,
        grid=(NP // RB,),
        in_specs=[
            pl.BlockSpec((NC, RB, D), lambda i: (0, i, 0)),
            pl.BlockSpec((NC, RB, 128), lambda i: (0, i, 0)),
            pl.BlockSpec((1, D), lambda i: (0, 0)),
            rspec, rspec, rspec, rspec, rspec,
        ],
        out_specs=[rspec] * 3,
        out_shape=[sds, sds, sds],
    )(s2, deg, b, x_pad, hn1, hn2, a1, a2)


def kernel(x, edge_index, W1, b1, W2, b2, W3, b3):
    info = plsc.get_sparse_core_info()
    if (info.num_cores, info.num_subcores, info.num_lanes) != (NC, NS, 16):
        raise ValueError(f"unexpected SC geometry: {info}")
    N, D = x.shape
    E = edge_index.shape[1]
    gran = NS * C
    NP = -(-(N + 1) // RB) * RB
    if NP % gran:
        NP = -(-NP // gran) * gran
    rpt = NP // NS
    K = 2 * (-(-E // (NW * C * 2)))
    Ep = NW * K * C

    row = edge_index[0]
    col = edge_index[1]
    pad = Ep - E
    if pad:
        padv = jnp.full((pad,), N, jnp.int32)
        row = jnp.concatenate([row, padv])
        col = jnp.concatenate([col, padv])
    packed = jnp.bitwise_or(row, jnp.left_shift(col, 16)).reshape(NW, K, C)
    x_pad = jnp.pad(x, ((0, NP - N), (0, 0)))
    b1r, b2r, b3r = (b.reshape(1, D) for b in (b1, b2, b3))

    deg = _make_deg(NP, K, rpt)(packed)
    agg = _make_agg(NP, D, K, rpt)

    hh1 = _tc_pre(x_pad, W1, deg, NP, D)
    s1 = agg(hh1, packed)
    a1, hn1, hh2 = _tc_mid(s1, deg, b1r, W2, NP, D)
    s2 = agg(hh2, packed)
    a2, hn2, hh3 = _tc_mid(s2, deg, b2r, W3, NP, D)
    s3 = agg(hh3, packed)
    a3, res_p, fin_p = _tc_post(s3, deg, b3r, x_pad, hn1, hn2, a1, a2, NP, D)
    return (res_p[:N], fin_p[:N], a3[:N])


# speedup vs baseline: 17.5546x; 3.3935x over previous
import functools

import jax
import jax.numpy as jnp
from jax import lax
from jax.experimental import pallas as pl
from jax.experimental.pallas import tpu as pltpu
from jax.experimental.pallas import tpu_sc as plsc

NC = 2
NS = 16
NW = NC * NS
C = 128
RB = 512


def _sc_mesh():
    return plsc.VectorSubcoreMesh(
        core_axis_name="c", subcore_axis_name="s",
        num_cores=NC, num_subcores=NS)


def _unpack_chunk(pk_v, j, row_c, col_c):
    for t in range(C // 16):
        v = pk_v[j, pl.ds(t * 16, 16)]
        if row_c is not None:
            row_c[pl.ds(t * 16, 16)] = jnp.bitwise_and(v, 0xFFFF)
        col_c[pl.ds(t * 16, 16)] = lax.shift_right_logical(v, 16)


def _make_deg(NP, K, rpt):
    nz = rpt // C

    @functools.partial(
        pl.kernel,
        out_type=jax.ShapeDtypeStruct((NC, NP, 128), jnp.float32),
        mesh=_sc_mesh(),
        scratch_types=[
            pltpu.VMEM((K, C), jnp.int32),
            pltpu.VMEM((C,), jnp.int32),
            pltpu.VMEM((C, 128), jnp.float32),
            pltpu.VMEM((C, 128), jnp.float32),
            pltpu.VMEM_SHARED((NP, 128), jnp.float32),
        ],
    )
    def deg_k(pk_hbm, deg_hbm, pk_v, col_c, ones_v, wb_v, acc):
        cid = lax.axis_index("c")
        sid = lax.axis_index("s")
        wid = sid * NC + cid

        def fill(i, _):
            for jj in range(8):
                ones_v[i, pl.ds(jj * 16, 16)] = jnp.ones((16,), jnp.float32)
                wb_v[i, pl.ds(jj * 16, 16)] = jnp.zeros((16,), jnp.float32)
            return 0
        lax.fori_loop(0, C, fill, 0)

        def zcopy(i, _):
            pltpu.sync_copy(wb_v, acc.at[pl.ds(sid * rpt + i * C, C)])
            return 0
        lax.fori_loop(0, nz, zcopy, 0)
        pltpu.sync_copy(pk_hbm.at[wid], pk_v)
        plsc.subcore_barrier()

        def body(j, _):
            _unpack_chunk(pk_v, j, None, col_c)
            pltpu.sync_copy(ones_v, acc.at[col_c], add=True)
            return 0
        lax.fori_loop(0, K, body, 0)
        plsc.subcore_barrier()

        def ocopy(i, _):
            base = sid * rpt + i * C
            pltpu.sync_copy(acc.at[pl.ds(base, C)], wb_v)
            pltpu.sync_copy(wb_v, deg_hbm.at[cid, pl.ds(base, C)])
            return 0
        lax.fori_loop(0, nz, ocopy, 0)

    return deg_k


def _make_agg(NP, D, K, rpt):
    nz = rpt // C

    @functools.partial(
        pl.kernel,
        out_type=jax.ShapeDtypeStruct((NC, NP, D), jnp.float32),
        mesh=_sc_mesh(),
        scratch_types=[
            pltpu.VMEM((K, C), jnp.int32),
            pltpu.VMEM((C,), jnp.int32),
            pltpu.VMEM((C,), jnp.int32),
            pltpu.VMEM((C,), jnp.int32),
            pltpu.VMEM((C,), jnp.int32),
            pltpu.VMEM((C, D), jnp.float32),
            pltpu.VMEM((C, D), jnp.float32),
            pltpu.VMEM_SHARED((NP, D), jnp.float32),
            pltpu.SemaphoreType.DMA,
            pltpu.SemaphoreType.DMA,
        ],
    )
    def agg_k(hhat_hbm, pk_hbm, out_hbm,
              pk_v, row_c0, col_c0, row_c1, col_c1, gb0, gb1,
              acc, sem0, sem1):
        cid = lax.axis_index("c")
        sid = lax.axis_index("s")
        wid = sid * NC + cid
        rows = (row_c0, row_c1)
        cols = (col_c0, col_c1)
        gbufs = (gb0, gb1)
        sems = (sem0, sem1)

        def zrow(i, _):
            for jj in range(D // 16):
                gb0[i, pl.ds(jj * 16, 16)] = jnp.zeros((16,), jnp.float32)
            return 0
        lax.fori_loop(0, C, zrow, 0)

        def zcopy(i, _):
            pltpu.sync_copy(gb0, acc.at[pl.ds(sid * rpt + i * C, C)])
            return 0
        lax.fori_loop(0, nz, zcopy, 0)

        pltpu.sync_copy(pk_hbm.at[wid], pk_v)
        plsc.subcore_barrier()

        for b in (0, 1):
            _unpack_chunk(pk_v, b, rows[b], cols[b])
            pltpu.async_copy(hhat_hbm.at[rows[b]], gbufs[b], sems[b])

        def body(i, _):
            for b in (0, 1):
                j = 2 * i + b
                pltpu.make_async_copy(hhat_hbm.at[rows[b]], gbufs[b],
                                      sems[b]).wait()
                pltpu.sync_copy(gbufs[b], acc.at[cols[b]], add=True)
                _unpack_chunk(pk_v, j + 2, rows[b], cols[b])
                pltpu.async_copy(hhat_hbm.at[rows[b]], gbufs[b], sems[b])
            return 0
        lax.fori_loop(0, K // 2 - 1, body, 0)

        for b in (0, 1):
            pltpu.make_async_copy(hhat_hbm.at[rows[b]], gbufs[b],
                                  sems[b]).wait()
            pltpu.sync_copy(gbufs[b], acc.at[cols[b]], add=True)
        plsc.subcore_barrier()

        def ocopy(i, _):
            base = sid * rpt + i * C
            pltpu.sync_copy(acc.at[pl.ds(base, C)], gb1)
            pltpu.sync_copy(gb1, out_hbm.at[cid, pl.ds(base, C)])
            return 0
        lax.fori_loop(0, nz, ocopy, 0)

    return agg_k


def _tc_mm(x_pad, W, NP, D):
    def body(xr, wr, out):
        out[...] = jnp.dot(xr[...], wr[...],
                           preferred_element_type=jnp.float32)

    return pl.pallas_call(
        body,
        grid=(NP // RB,),
        in_specs=[
            pl.BlockSpec((RB, D), lambda i: (i, 0)),
            pl.BlockSpec((D, D), lambda i: (0, 0)),
        ],
        out_specs=pl.BlockSpec((RB, D), lambda i: (i, 0)),
        out_shape=jax.ShapeDtypeStruct((NP, D), jnp.float32),
    )(x_pad, W)


def _tc_scale(t1, deg, NP, D):
    def body(tr, dr, hh_out, dinv_out):
        d = dr[0, :, 0:1] + dr[1, :, 0:1]
        dinv = jnp.where(d > 0, lax.rsqrt(jnp.where(d > 0, d, 1.0)), 0.0)
        dinv_out[...] = dinv
        hh_out[...] = dinv * tr[...]

    return pl.pallas_call(
        body,
        grid=(NP // RB,),
        in_specs=[
            pl.BlockSpec((RB, D), lambda i: (i, 0)),
            pl.BlockSpec((NC, RB, 128), lambda i: (0, i, 0)),
        ],
        out_specs=[pl.BlockSpec((RB, D), lambda i: (i, 0)),
                   pl.BlockSpec((RB, 1), lambda i: (i, 0))],
        out_shape=[jax.ShapeDtypeStruct((NP, D), jnp.float32),
                   jax.ShapeDtypeStruct((NP, 1), jnp.float32)],
    )(t1, deg)


def _tc_mid(s2, dinv_col, b, Wn, NP, D):
    def body(sr, dr, br, wr, a_out, hn_out, hh_out):
        dinv = dr[...]
        a = dinv * (sr[0] + sr[1]) + br[...]
        a_out[...] = a
        nsq = jnp.sum(a * a, axis=-1, keepdims=True)
        hn = a * lax.rsqrt(jnp.maximum(nsq, 1e-24))
        hn_out[...] = hn
        hh_out[...] = dinv * jnp.dot(hn, wr[...],
                                     preferred_element_type=jnp.float32)

    sds = jax.ShapeDtypeStruct((NP, D), jnp.float32)
    return pl.pallas_call(
        body,
        grid=(NP // RB,),
        in_specs=[
            pl.BlockSpec((NC, RB, D), lambda i: (0, i, 0)),
            pl.BlockSpec((RB, 1), lambda i: (i, 0)),
            pl.BlockSpec((1, D), lambda i: (0, 0)),
            pl.BlockSpec((D, D), lambda i: (0, 0)),
        ],
        out_specs=[pl.BlockSpec((RB, D), lambda i: (i, 0))] * 3,
        out_shape=[sds, sds, sds],
    )(s2, dinv_col, b, Wn)


def _tc_post(s2, dinv_col, b, x_pad, hn1, hn2, a1, a2, NP, D):
    def body(sr, dr, br, xr, h1r, h2r, a1r, a2r, a3_out, res_out, fin_out):
        dinv = dr[...]
        a3 = dinv * (sr[0] + sr[1]) + br[...]
        a3_out[...] = a3
        nsq = jnp.sum(a3 * a3, axis=-1, keepdims=True)
        hn3 = a3 * lax.rsqrt(jnp.maximum(nsq, 1e-24))
        res_out[...] = (xr[...] + h1r[...] + 0.5 * h2r[...]
                        + (1.0 / 3.0) * hn3)
        fin_out[...] = (a1r[...] + a2r[...] + a3) * (1.0 / 3.0)

    sds = jax.ShapeDtypeStruct((NP, D), jnp.float32)
    rspec = pl.BlockSpec((RB, D), lambda i: (i, 0))
    return pl.pallas_call(
        body,
        grid=(NP // RB,),
        in_specs=[
            pl.BlockSpec((NC, RB, D), lambda i: (0, i, 0)),
            pl.BlockSpec((RB, 1), lambda i: (i, 0)),
            pl.BlockSpec((1, D), lambda i: (0, 0)),
            rspec, rspec, rspec, rspec, rspec,
        ],
        out_specs=[rspec] * 3,
        out_shape=[sds, sds, sds],
    )(s2, dinv_col, b, x_pad, hn1, hn2, a1, a2)


def kernel(x, edge_index, W1, b1, W2, b2, W3, b3):
    info = plsc.get_sparse_core_info()
    if (info.num_cores, info.num_subcores, info.num_lanes) != (NC, NS, 16):
        raise ValueError(f"unexpected SC geometry: {info}")
    N, D = x.shape
    E = edge_index.shape[1]
    gran = NS * C
    NP = -(-(N + 1) // RB) * RB
    if NP % gran:
        NP = -(-NP // gran) * gran
    rpt = NP // NS
    K = 2 * (-(-E // (NW * C * 2)))
    Ep = NW * K * C

    row = edge_index[0]
    col = edge_index[1]
    pad = Ep - E
    if pad:
        padv = N + (jnp.arange(pad, dtype=jnp.int32) % (NP - N))
        row = jnp.concatenate([row, padv])
        col = jnp.concatenate([col, padv])
    packed = jnp.bitwise_or(row, jnp.left_shift(col, 16)).reshape(NW, K, C)
    x_pad = jnp.pad(x, ((0, NP - N), (0, 0)))
    b1r, b2r, b3r = (b.reshape(1, D) for b in (b1, b2, b3))

    t1 = _tc_mm(x_pad, W1, NP, D)
    deg = _make_deg(NP, K, rpt)(packed)
    agg = _make_agg(NP, D, K, rpt)
    hh1, dinv_col = _tc_scale(t1, deg, NP, D)
    s1 = agg(hh1, packed)
    a1, hn1, hh2 = _tc_mid(s1, dinv_col, b1r, W2, NP, D)
    s2 = agg(hh2, packed)
    a2, hn2, hh3 = _tc_mid(s2, dinv_col, b2r, W3, NP, D)
    s3 = agg(hh3, packed)
    a3, res_p, fin_p = _tc_post(s3, dinv_col, b3r, x_pad, hn1, hn2, a1, a2, NP, D)
    return (res_p[:N], fin_p[:N], a3[:N])
